# baseline (speedup 1.0000x reference)
import jax
import jax.numpy as jnp
from jax.experimental import pallas as pl

N_USERS = 5000
N_GAMES = 5000
D = 128
H = 128


def _sage(x_src, x_dst, src, dst, Wl, bl, Wr, n_dst):
    s = jax.ops.segment_sum(x_src[src], dst, num_segments=n_dst)
    cnt = jax.ops.segment_sum(jnp.ones((src.shape[0],), x_src.dtype), dst, num_segments=n_dst)
    mean = s / jnp.clip(cnt, 1.0)[:, None]
    return mean @ Wl.T + bl + x_dst @ Wr.T


def _dec_body(z_ref, w1_ref, b1_ref, w2_ref, b2_ref, o_ref):
    h = jnp.maximum(z_ref[...] @ w1_ref[...].T + b1_ref[...][None, :], 0.0)
    o_ref[...] = ((h @ w2_ref[...].T)[:, 0] + b2_ref[...][0])[None, None, :]


def kernel(x_user, x_game, edge_index_plays, edge_index_rev, edge_score_index, params):
    u, g = x_user, x_game
    for l in (1, 2, 3):
        u_new = _sage(g, u, edge_index_rev[0], edge_index_rev[1],
                      params['W%dr_l' % l], params['b%dr' % l], params['W%dr_r' % l], N_USERS)
        g_new = _sage(u, g, edge_index_plays[0], edge_index_plays[1],
                      params['W%dp_l' % l], params['b%dp' % l], params['W%dp_r' % l], N_GAMES)
        if l < 3:
            u_new = jax.nn.relu(u_new)
            g_new = jax.nn.relu(g_new)
        u, g = u_new, g_new

    row, col = edge_score_index[0], edge_score_index[1]
    z = jnp.concatenate([u[row], g[col]], axis=-1)
    n = z.shape[0]
    blk = 2000
    out = pl.pallas_call(
        _dec_body,
        grid=(n // blk,),
        in_specs=[
            pl.BlockSpec((blk, 2 * H), lambda i: (i, 0)),
            pl.BlockSpec((H, 2 * H), lambda i: (0, 0)),
            pl.BlockSpec((H,), lambda i: (0,)),
            pl.BlockSpec((1, H), lambda i: (0, 0)),
            pl.BlockSpec((1,), lambda i: (0,)),
        ],
        out_specs=pl.BlockSpec((1, 1, blk), lambda i: (i, 0, 0)),
        out_shape=jax.ShapeDtypeStruct((n // blk, 1, blk), jnp.float32),
    )(z, params['Wd1'], params['bd1'], params['Wd2'], params['bd2'])
    return out.reshape(-1)


# speedup vs baseline: 1.3627x; 1.3627x over previous
import functools

import jax
import jax.numpy as jnp
from jax import lax
from jax.experimental import pallas as pl
from jax.experimental.pallas import tpu as pltpu
from jax.experimental.pallas import tpu_sc as plsc

N_USERS = 5000
N_GAMES = 5000
N = 5000
D = 128
H = 128
E = 320000
NSCORE = 100000

NSL = 8
LANES = 16
SEG_C = 1000
SEG_HALF = E // 2
SEG_NCH = SEG_HALF // SEG_C

CNT_C = 2000
CNT_PER_TILE = E // 32
CNT_NCH = CNT_PER_TILE // CNT_C

DEC_C = 400
DEC_NCH = NSCORE // DEC_C


def _seg_mesh():
    return plsc.VectorSubcoreMesh(core_axis_name="c", subcore_axis_name="s")


def _make_counts():
    def body(u_hbm, g_hbm, out_hbm, ubuf, gbuf, ubuf_f, cnt, shared, sem):
        c = lax.axis_index("c")
        s = lax.axis_index("s")
        wid = c * 16 + s
        ones = jnp.ones((LANES,), jnp.float32)

        def zbody(i, _):
            cnt[pl.ds(i * LANES, LANES)] = jnp.zeros((LANES,), jnp.float32)
            return 0
        lax.fori_loop(0, 10000 // LANES, zbody, 0)

        base_tile = wid * CNT_PER_TILE

        def chunk(k, _):
            b = base_tile + k * CNT_C
            pltpu.sync_copy(u_hbm.at[pl.ds(b, CNT_C)], ubuf)
            pltpu.sync_copy(g_hbm.at[pl.ds(b, CNT_C)], gbuf)

            def grp(j, _):
                uvec = ubuf[pl.ds(j * LANES, LANES)]
                gvec = gbuf[pl.ds(j * LANES, LANES)]
                plsc.addupdate_scatter(cnt, [uvec], ones)
                plsc.addupdate_scatter(cnt, [gvec + 5000], ones)
                return 0
            lax.fori_loop(0, CNT_C // LANES, grp, 0)
            return 0
        lax.fori_loop(0, CNT_NCH, chunk, 0)

        pltpu.sync_copy(cnt, shared.at[s])
        plsc.subcore_barrier()

        @pl.when(s == 0)
        def _():
            def partner(p, _):
                def pchunk(q, _):
                    pltpu.sync_copy(shared.at[p, pl.ds(q * 2000, 2000)], ubuf_f)

                    def radd(r, _):
                        plsc.addupdate(
                            cnt.at[pl.ds(q * 2000 + r * LANES, LANES)],
                            ubuf_f[pl.ds(r * LANES, LANES)])
                        return 0
                    lax.fori_loop(0, 2000 // LANES, radd, 0)
                    return 0
                lax.fori_loop(0, 5, pchunk, 0)
                return 0
            lax.fori_loop(1, 16, partner, 0)
            pltpu.sync_copy(cnt, out_hbm.at[c])
        del sem

    return pl.kernel(
        body,
        out_type=jax.ShapeDtypeStruct((2, 10000), jnp.float32),
        mesh=_seg_mesh(),
        compiler_params=pltpu.CompilerParams(use_tc_tiling_on_sc=False, needs_layout_passes=False),
        scratch_types=[
            pltpu.VMEM((CNT_C,), jnp.int32),
            pltpu.VMEM((CNT_C,), jnp.int32),
            pltpu.VMEM((2000,), jnp.float32),
            pltpu.VMEM((10000,), jnp.float32),
            pltpu.VMEM_SHARED((16, 10000), jnp.float32),
            pltpu.SemaphoreType.DMA,
        ],
    )


def _segsum_body(xg_hbm, xu_hbm, src_rev, dst_rev, src_pl, dst_pl, out_hbm,
                 sbuf0, sbuf1, dbuf0, dbuf1, gath0, gath1, acc, shared,
                 sem_i0, sem_i1, sem_g0, sem_g1):
    c = lax.axis_index("c")
    s = lax.axis_index("s")
    slc = s // 2
    half = s % 2
    lane = lax.iota(jnp.int32, LANES)
    row_pat = lane % 8
    col_lo = lane // 8

    def zbody(i, _):
        acc[i] = jnp.zeros((LANES,), jnp.float32)
        return 0
    lax.fori_loop(0, N, zbody, 0)

    ebase0 = half * SEG_HALF

    def run_dir(table, src_sl, dst_e):
        sbufs = (sbuf0, sbuf1)
        dbufs = (dbuf0, dbuf1)
        gaths = (gath0, gath1)
        sems_i = (sem_i0, sem_i1)
        sems_g = (sem_g0, sem_g1)

        sbase = slc * E + half * SEG_HALF

        pltpu.sync_copy(src_sl.at[pl.ds(sbase, SEG_C)], sbuf0)
        pltpu.sync_copy(dst_e.at[pl.ds(ebase0, SEG_C)], dbuf0)
        g0 = pltpu.async_copy(table.at[sbuf0], gath0, sem_g0)
        i1a = pltpu.async_copy(src_sl.at[pl.ds(sbase + SEG_C, SEG_C)],
                               sbuf1, sem_i1)
        i1b = pltpu.async_copy(dst_e.at[pl.ds(ebase0 + SEG_C, SEG_C)],
                               dbuf1, sem_i1)
        del g0, i1a, i1b

        def accum(gath, dbuf):
            def grp(j, _):
                evec = row_pat + j * 8
                dvec = plsc.load_gather(dbuf, [evec])
                for fp in range(NSL):
                    colv = col_lo + 2 * fp
                    vals = plsc.load_gather(gath, [evec, colv])
                    plsc.addupdate_scatter(acc, [dvec, colv], vals)
                return 0
            lax.fori_loop(0, SEG_C // 8, grp, 0)

        def pair(kk, _):
            for b in (0, 1):
                k = kk * 2 + b
                bn = 1 - b
                @pl.when(k + 1 < SEG_NCH)
                def _():
                    pltpu.make_async_copy(
                        src_sl.at[pl.ds(0, SEG_C)], sbufs[bn],
                        sems_i[bn]).wait()
                    pltpu.make_async_copy(
                        dst_e.at[pl.ds(0, SEG_C)], dbufs[bn],
                        sems_i[bn]).wait()
                    pltpu.async_copy(table.at[sbufs[bn]], gaths[bn],
                                     sems_g[bn])
                pltpu.make_async_copy(table.at[sbufs[b]], gaths[b],
                                      sems_g[b]).wait()
                accum(gaths[b], dbufs[b])
                @pl.when(k + 2 < SEG_NCH)
                def _():
                    eb = ebase0 + (k + 2) * SEG_C
                    pltpu.async_copy(src_sl.at[pl.ds(sbase + (k + 2) * SEG_C,
                                                     SEG_C)],
                                     sbufs[b], sems_i[b])
                    pltpu.async_copy(dst_e.at[pl.ds(eb, SEG_C)],
                                     dbufs[b], sems_i[b])
            return 0
        lax.fori_loop(0, SEG_NCH // 2, pair, 0)

    @pl.when(c == 0)
    def _():
        run_dir(xg_hbm, src_rev, dst_rev)

    @pl.when(c == 1)
    def _():
        run_dir(xu_hbm, src_pl, dst_pl)

    def pchunk(q, _):
        @pl.when(half == 1)
        def _():
            pltpu.sync_copy(acc.at[pl.ds(q * SEG_C, SEG_C)], shared.at[slc])
        plsc.subcore_barrier()

        @pl.when(half == 0)
        def _():
            pltpu.sync_copy(shared.at[slc], gath0)

            def radd(r, _):
                plsc.addupdate(acc.at[q * SEG_C + r], gath0[r])
                return 0
            lax.fori_loop(0, SEG_C, radd, 0)
        plsc.subcore_barrier()
        return 0
    lax.fori_loop(0, N // SEG_C, pchunk, 0)

    @pl.when(half == 0)
    def _():
        pltpu.sync_copy(acc, out_hbm.at[c, :, slc, :])


def _make_segsum():
    return pl.kernel(
        _segsum_body,
        out_type=jax.ShapeDtypeStruct((2, N, NSL, LANES), jnp.float32),
        mesh=_seg_mesh(),
        compiler_params=pltpu.CompilerParams(use_tc_tiling_on_sc=False, needs_layout_passes=False),
        scratch_types=[
            pltpu.VMEM((SEG_C,), jnp.int32),
            pltpu.VMEM((SEG_C,), jnp.int32),
            pltpu.VMEM((SEG_C,), jnp.int32),
            pltpu.VMEM((SEG_C,), jnp.int32),
            pltpu.VMEM((SEG_C, LANES), jnp.float32),
            pltpu.VMEM((SEG_C, LANES), jnp.float32),
            pltpu.VMEM((N, LANES), jnp.float32),
            pltpu.VMEM_SHARED((NSL, SEG_C, LANES), jnp.float32),
            pltpu.SemaphoreType.DMA,
            pltpu.SemaphoreType.DMA,
            pltpu.SemaphoreType.DMA,
            pltpu.SemaphoreType.DMA,
        ],
    )


def _decoder_body(p_hbm, q_hbm, row_hbm, col_hbm, bd1_hbm, wd2_hbm, bd2_hbm,
                  out_hbm, rbuf, cbuf, buf, obuf, bd1v, wd2v, bd2v, sem):
    c = lax.axis_index("c")
    s = lax.axis_index("s")
    wid = s * 2 + c
    lane = lax.iota(jnp.int32, LANES)
    last_mask = lane == (LANES - 1)

    pltpu.sync_copy(bd1_hbm, bd1v)
    pltpu.sync_copy(wd2_hbm, wd2v)
    pltpu.sync_copy(bd2_hbm, bd2v)

    bd2vec = bd2v[pl.ds(0, LANES)]
    b1 = [bd1v[pl.ds(f * LANES, LANES)] for f in range(NSL)]
    w2 = [wd2v[pl.ds(f * LANES, LANES)] for f in range(NSL)]

    def chunk(k, _):
        jc = wid + 32 * k

        @pl.when(jc < DEC_NCH)
        def _():
            b = jc * DEC_C
            pltpu.sync_copy(row_hbm.at[pl.ds(b, DEC_C)], rbuf)
            pltpu.sync_copy(col_hbm.at[pl.ds(b, DEC_C)], cbuf)
            pltpu.async_copy(p_hbm.at[rbuf], buf, sem).wait()
            pltpu.async_copy(q_hbm.at[cbuf], buf, sem, add=True).wait()

            def edge(e, _):
                acc = bd2vec
                for f in range(NSL):
                    v = buf[e, pl.ds(f * LANES, LANES)]
                    acc = acc + jnp.maximum(v + b1[f], 0.0) * w2[f]
                tot = plsc.cumsum(acc)
                plsc.store_scatter(obuf, [jnp.broadcast_to(e, (LANES,))],
                                   tot, mask=last_mask)
                return 0
            lax.fori_loop(0, DEC_C, edge, 0)
            pltpu.sync_copy(obuf, out_hbm.at[pl.ds(b, DEC_C)])
        return 0

    lax.fori_loop(0, (DEC_NCH + 31) // 32, chunk, 0)


def _make_decoder():
    return pl.kernel(
        _decoder_body,
        out_type=jax.ShapeDtypeStruct((NSCORE,), jnp.float32),
        mesh=_seg_mesh(),
        compiler_params=pltpu.CompilerParams(use_tc_tiling_on_sc=False, needs_layout_passes=False),
        scratch_types=[
            pltpu.VMEM((DEC_C,), jnp.int32),
            pltpu.VMEM((DEC_C,), jnp.int32),
            pltpu.VMEM((DEC_C, D), jnp.float32),
            pltpu.VMEM((DEC_C,), jnp.float32),
            pltpu.VMEM((D,), jnp.float32),
            pltpu.VMEM((D,), jnp.float32),
            pltpu.VMEM((LANES,), jnp.float32),
            pltpu.SemaphoreType.DMA,
        ],
    )


_BLK = 1000


def _mm_t(a, w):
    return lax.dot_general(a, w, (((1,), (1,)), ((), ())),
                           preferred_element_type=jnp.float32)


def _update_body(relu, sum_ref, cnt_ref, x_ref, wl_ref, b_ref, wr_ref, o_ref):
    cnt = cnt_ref[0, 0, 0] + cnt_ref[1, 0, 0]
    mean = sum_ref[...] / jnp.maximum(cnt, 1.0)[:, None]
    y = _mm_t(mean, wl_ref[...]) + _mm_t(x_ref[...], wr_ref[...]) \
        + b_ref[...][None, :]
    if relu:
        y = jnp.maximum(y, 0.0)
    o_ref[...] = y


def _update_dec_body(sum_ref, cnt_ref, x_ref, wl_ref, b_ref, wr_ref, wd_ref,
                     o_ref):
    cnt = cnt_ref[0, 0, 0] + cnt_ref[1, 0, 0]
    mean = sum_ref[...] / jnp.maximum(cnt, 1.0)[:, None]
    y = _mm_t(mean, wl_ref[...]) + _mm_t(x_ref[...], wr_ref[...]) \
        + b_ref[...][None, :]
    o_ref[...] = _mm_t(y, wd_ref[...])


def _full(shape):
    return pl.BlockSpec(shape, lambda i: tuple(0 for _ in shape))


def _update_tc(sum_x, cnt2, x, wl, b, wr, relu):
    cnt2 = cnt2.reshape(2, N // _BLK, 1, _BLK)
    return pl.pallas_call(
        functools.partial(_update_body, relu),
        grid=(N // _BLK,),
        in_specs=[
            pl.BlockSpec((_BLK, D), lambda i: (i, 0)),
            pl.BlockSpec((2, 1, 1, _BLK), lambda i: (0, i, 0, 0)),
            pl.BlockSpec((_BLK, D), lambda i: (i, 0)),
            _full((H, D)), _full((H,)), _full((H, D)),
        ],
        out_specs=pl.BlockSpec((_BLK, H), lambda i: (i, 0)),
        out_shape=jax.ShapeDtypeStruct((N, H), jnp.float32),
    )(sum_x, cnt2, x, wl, b, wr)


def _update_dec_tc(sum_x, cnt2, x, wl, b, wr, wd):
    cnt2 = cnt2.reshape(2, N // _BLK, 1, _BLK)
    return pl.pallas_call(
        _update_dec_body,
        grid=(N // _BLK,),
        in_specs=[
            pl.BlockSpec((_BLK, D), lambda i: (i, 0)),
            pl.BlockSpec((2, 1, 1, _BLK), lambda i: (0, i, 0, 0)),
            pl.BlockSpec((_BLK, D), lambda i: (i, 0)),
            _full((H, D)), _full((H,)), _full((H, D)), _full((H, H)),
        ],
        out_specs=pl.BlockSpec((_BLK, H), lambda i: (i, 0)),
        out_shape=jax.ShapeDtypeStruct((N, H), jnp.float32),
    )(sum_x, cnt2, x, wl, b, wr, wd)


def kernel(x_user, x_game, edge_index_plays, edge_index_rev, edge_score_index,
           params):
    del edge_index_rev
    pu = edge_index_plays[0].astype(jnp.int32)
    pg = edge_index_plays[1].astype(jnp.int32)
    sl8 = jnp.arange(8, dtype=jnp.int32)[:, None]
    src_rev = (pg[None, :] * 8 + sl8).reshape(-1)
    src_pl = (pu[None, :] * 8 + sl8).reshape(-1)
    dst_rev = pu
    dst_pl = pg

    counts = _make_counts()(pu, pg)
    cnt_u2 = counts[:, :5000]
    cnt_g2 = counts[:, 5000:]

    segsum = _make_segsum()
    u, g = x_user, x_game
    for l in (1, 2, 3):
        sums = segsum(g.reshape(N * NSL, LANES), u.reshape(N * NSL, LANES),
                      src_rev, dst_rev, src_pl, dst_pl)
        sum_u = sums[0].reshape(N, D)
        sum_g = sums[1].reshape(N, D)
        if l < 3:
            u_new = _update_tc(sum_u, cnt_u2, u, params['W%dr_l' % l],
                               params['b%dr' % l], params['W%dr_r' % l], True)
            g_new = _update_tc(sum_g, cnt_g2, g, params['W%dp_l' % l],
                               params['b%dp' % l], params['W%dp_r' % l], True)
            u, g = u_new, g_new
        else:
            p = _update_dec_tc(sum_u, cnt_u2, u, params['W3r_l'],
                               params['b3r'], params['W3r_r'],
                               params['Wd1'][:, :H])
            q = _update_dec_tc(sum_g, cnt_g2, g, params['W3p_l'],
                               params['b3p'], params['W3p_r'],
                               params['Wd1'][:, H:])

    row = edge_score_index[0].astype(jnp.int32)
    col = edge_score_index[1].astype(jnp.int32)
    bd2p = jnp.zeros((LANES,), jnp.float32).at[0].set(params['bd2'][0])
    out = _make_decoder()(p, q, row, col, params['bd1'],
                          params['Wd2'].reshape(-1), bd2p)
    return out


# speedup vs baseline: 2.4814x; 1.8210x over previous
import functools

import jax
import jax.numpy as jnp
from jax import lax
from jax.experimental import pallas as pl
from jax.experimental.pallas import tpu as pltpu
from jax.experimental.pallas import tpu_sc as plsc

N_USERS = 5000
N_GAMES = 5000
N = 5000
D = 128
H = 128
E = 320000
NSCORE = 100000

NSL = 8
LANES = 16
SEG_C = 1000
SEG_HALF = E // 2
SEG_NCH = SEG_HALF // SEG_C

CNT_C = 2000
CNT_PER_TILE = E // 32
CNT_NCH = CNT_PER_TILE // CNT_C

DEC_C = 400
DEC_NCH = NSCORE // DEC_C


def _seg_mesh():
    return plsc.VectorSubcoreMesh(core_axis_name="c", subcore_axis_name="s")


def _make_counts():
    def body(u_hbm, g_hbm, out_hbm, ubuf, gbuf, ubuf_f, cnt, shared, sem):
        c = lax.axis_index("c")
        s = lax.axis_index("s")
        wid = c * 16 + s
        ones = jnp.ones((LANES,), jnp.float32)

        @plsc.parallel_loop(0, 10000 // LANES, 1, unroll=8)
        def zbody(i):
            cnt[pl.ds(i * LANES, LANES)] = jnp.zeros((LANES,), jnp.float32)

        base_tile = wid * CNT_PER_TILE

        def chunk(k, _):
            b = base_tile + k * CNT_C
            pltpu.sync_copy(u_hbm.at[pl.ds(b, CNT_C)], ubuf)
            pltpu.sync_copy(g_hbm.at[pl.ds(b, CNT_C)], gbuf)

            @plsc.parallel_loop(0, CNT_C // LANES, 1, unroll=4)
            def grp(j):
                uvec = ubuf[pl.ds(j * LANES, LANES)]
                gvec = gbuf[pl.ds(j * LANES, LANES)]
                plsc.addupdate_scatter(cnt, [uvec], ones)
                plsc.addupdate_scatter(cnt, [gvec + 5000], ones)
            return 0
        lax.fori_loop(0, CNT_NCH, chunk, 0)

        pltpu.sync_copy(cnt, shared.at[s])
        plsc.subcore_barrier()

        @pl.when(s == 0)
        def _():
            def partner(p, _):
                def pchunk(q, _):
                    pltpu.sync_copy(shared.at[p, pl.ds(q * 2000, 2000)], ubuf_f)

                    qb = q * 2000

                    @plsc.parallel_loop(0, 2000 // LANES, 1, unroll=8)
                    def radd(r):
                        plsc.addupdate(
                            cnt.at[pl.ds(qb + r * LANES, LANES)],
                            ubuf_f[pl.ds(r * LANES, LANES)])
                    return 0
                lax.fori_loop(0, 5, pchunk, 0)
                return 0
            lax.fori_loop(1, 16, partner, 0)
            pltpu.sync_copy(cnt, out_hbm.at[c])
        del sem

    return pl.kernel(
        body,
        out_type=jax.ShapeDtypeStruct((2, 10000), jnp.float32),
        mesh=_seg_mesh(),
        compiler_params=pltpu.CompilerParams(use_tc_tiling_on_sc=False, needs_layout_passes=False),
        scratch_types=[
            pltpu.VMEM((CNT_C,), jnp.int32),
            pltpu.VMEM((CNT_C,), jnp.int32),
            pltpu.VMEM((2000,), jnp.float32),
            pltpu.VMEM((10000,), jnp.float32),
            pltpu.VMEM_SHARED((16, 10000), jnp.float32),
            pltpu.SemaphoreType.DMA,
        ],
    )


def _segsum_body(xg_hbm, xu_hbm, src_rev, dst_rev, src_pl, dst_pl, out_hbm,
                 sbuf0, sbuf1, dbuf0, dbuf1, gath0, gath1, acc, shared,
                 sem_i0, sem_i1, sem_g0, sem_g1):
    c = lax.axis_index("c")
    s = lax.axis_index("s")
    slc = s // 2
    half = s % 2
    lane = lax.iota(jnp.int32, LANES)
    row_pat = lane % 8
    col_lo = lane // 8

    @plsc.parallel_loop(0, N, 1, unroll=8)
    def zbody(i):
        acc[i] = jnp.zeros((LANES,), jnp.float32)

    ebase0 = half * SEG_HALF

    def run_dir(table, src_sl, dst_e):
        sbufs = (sbuf0, sbuf1)
        dbufs = (dbuf0, dbuf1)
        gaths = (gath0, gath1)
        sems_i = (sem_i0, sem_i1)
        sems_g = (sem_g0, sem_g1)

        sbase = slc * E + half * SEG_HALF

        pltpu.sync_copy(src_sl.at[pl.ds(sbase, SEG_C)], sbuf0)
        pltpu.sync_copy(dst_e.at[pl.ds(ebase0, SEG_C)], dbuf0)
        g0 = pltpu.async_copy(table.at[sbuf0], gath0, sem_g0)
        i1a = pltpu.async_copy(src_sl.at[pl.ds(sbase + SEG_C, SEG_C)],
                               sbuf1, sem_i1)
        i1b = pltpu.async_copy(dst_e.at[pl.ds(ebase0 + SEG_C, SEG_C)],
                               dbuf1, sem_i1)
        del g0, i1a, i1b

        def accum(gath, dbuf):
            @plsc.parallel_loop(0, SEG_C // 8, 1, unroll=2)
            def grp(j):
                evec = row_pat + j * 8
                dvec = plsc.load_gather(dbuf, [evec])
                for fp in range(NSL):
                    colv = col_lo + 2 * fp
                    vals = plsc.load_gather(gath, [evec, colv])
                    plsc.addupdate_scatter(acc, [dvec, colv], vals)

        def pair(kk, _):
            for b in (0, 1):
                k = kk * 2 + b
                bn = 1 - b
                @pl.when(k + 1 < SEG_NCH)
                def _():
                    pltpu.make_async_copy(
                        src_sl.at[pl.ds(0, SEG_C)], sbufs[bn],
                        sems_i[bn]).wait()
                    pltpu.make_async_copy(
                        dst_e.at[pl.ds(0, SEG_C)], dbufs[bn],
                        sems_i[bn]).wait()
                    pltpu.async_copy(table.at[sbufs[bn]], gaths[bn],
                                     sems_g[bn])
                pltpu.make_async_copy(table.at[sbufs[b]], gaths[b],
                                      sems_g[b]).wait()
                accum(gaths[b], dbufs[b])
                @pl.when(k + 2 < SEG_NCH)
                def _():
                    eb = ebase0 + (k + 2) * SEG_C
                    pltpu.async_copy(src_sl.at[pl.ds(sbase + (k + 2) * SEG_C,
                                                     SEG_C)],
                                     sbufs[b], sems_i[b])
                    pltpu.async_copy(dst_e.at[pl.ds(eb, SEG_C)],
                                     dbufs[b], sems_i[b])
            return 0
        lax.fori_loop(0, SEG_NCH // 2, pair, 0)

    @pl.when(c == 0)
    def _():
        run_dir(xg_hbm, src_rev, dst_rev)

    @pl.when(c == 1)
    def _():
        run_dir(xu_hbm, src_pl, dst_pl)

    def pchunk(q, _):
        @pl.when(half == 1)
        def _():
            pltpu.sync_copy(acc.at[pl.ds(q * SEG_C, SEG_C)], shared.at[slc])
        plsc.subcore_barrier()

        @pl.when(half == 0)
        def _():
            pltpu.sync_copy(shared.at[slc], gath0)

            qb = q * SEG_C

            @plsc.parallel_loop(0, SEG_C, 1, unroll=8)
            def radd(r):
                plsc.addupdate(acc.at[qb + r], gath0[r])
        plsc.subcore_barrier()
        return 0
    lax.fori_loop(0, N // SEG_C, pchunk, 0)

    @pl.when(half == 0)
    def _():
        pltpu.sync_copy(acc, out_hbm.at[c, :, slc, :])


def _make_segsum():
    return pl.kernel(
        _segsum_body,
        out_type=jax.ShapeDtypeStruct((2, N, NSL, LANES), jnp.float32),
        mesh=_seg_mesh(),
        compiler_params=pltpu.CompilerParams(use_tc_tiling_on_sc=False, needs_layout_passes=False),
        scratch_types=[
            pltpu.VMEM((SEG_C,), jnp.int32),
            pltpu.VMEM((SEG_C,), jnp.int32),
            pltpu.VMEM((SEG_C,), jnp.int32),
            pltpu.VMEM((SEG_C,), jnp.int32),
            pltpu.VMEM((SEG_C, LANES), jnp.float32),
            pltpu.VMEM((SEG_C, LANES), jnp.float32),
            pltpu.VMEM((N, LANES), jnp.float32),
            pltpu.VMEM_SHARED((NSL, SEG_C, LANES), jnp.float32),
            pltpu.SemaphoreType.DMA,
            pltpu.SemaphoreType.DMA,
            pltpu.SemaphoreType.DMA,
            pltpu.SemaphoreType.DMA,
        ],
    )


def _decoder_body(p_hbm, q_hbm, row_hbm, col_hbm, bd1_hbm, wd2_hbm, bd2_hbm,
                  out_hbm, rbuf, cbuf, buf, obuf, bd1v, wd2v, bd2v, sem):
    c = lax.axis_index("c")
    s = lax.axis_index("s")
    wid = s * 2 + c
    lane = lax.iota(jnp.int32, LANES)
    last_mask = lane == (LANES - 1)

    pltpu.sync_copy(bd1_hbm, bd1v)
    pltpu.sync_copy(wd2_hbm, wd2v)
    pltpu.sync_copy(bd2_hbm, bd2v)

    bd2vec = bd2v[pl.ds(0, LANES)]
    b1 = [bd1v[pl.ds(f * LANES, LANES)] for f in range(NSL)]
    w2 = [wd2v[pl.ds(f * LANES, LANES)] for f in range(NSL)]

    def chunk(k, _):
        jc = wid + 32 * k

        @pl.when(jc < DEC_NCH)
        def _():
            b = jc * DEC_C
            pltpu.sync_copy(row_hbm.at[pl.ds(b, DEC_C)], rbuf)
            pltpu.sync_copy(col_hbm.at[pl.ds(b, DEC_C)], cbuf)
            pltpu.async_copy(p_hbm.at[rbuf], buf, sem).wait()
            pltpu.async_copy(q_hbm.at[cbuf], buf, sem, add=True).wait()

            @plsc.parallel_loop(0, DEC_C, 1, unroll=2)
            def edge(e):
                acc = bd2vec
                for f in range(NSL):
                    v = buf[e, pl.ds(f * LANES, LANES)]
                    acc = acc + jnp.maximum(v + b1[f], 0.0) * w2[f]
                tot = plsc.cumsum(acc)
                plsc.store_scatter(obuf, [jnp.broadcast_to(e, (LANES,))],
                                   tot, mask=last_mask)
            pltpu.sync_copy(obuf, out_hbm.at[pl.ds(b, DEC_C)])
        return 0

    lax.fori_loop(0, (DEC_NCH + 31) // 32, chunk, 0)


def _make_decoder():
    return pl.kernel(
        _decoder_body,
        out_type=jax.ShapeDtypeStruct((NSCORE,), jnp.float32),
        mesh=_seg_mesh(),
        compiler_params=pltpu.CompilerParams(use_tc_tiling_on_sc=False, needs_layout_passes=False),
        scratch_types=[
            pltpu.VMEM((DEC_C,), jnp.int32),
            pltpu.VMEM((DEC_C,), jnp.int32),
            pltpu.VMEM((DEC_C, D), jnp.float32),
            pltpu.VMEM((DEC_C,), jnp.float32),
            pltpu.VMEM((D,), jnp.float32),
            pltpu.VMEM((D,), jnp.float32),
            pltpu.VMEM((LANES,), jnp.float32),
            pltpu.SemaphoreType.DMA,
        ],
    )


_BLK = 1000


def _mm_t(a, w):
    return lax.dot_general(a, w, (((1,), (1,)), ((), ())),
                           preferred_element_type=jnp.float32)


def _update_body(relu, sum_ref, cnt_ref, x_ref, wl_ref, b_ref, wr_ref, o_ref):
    cnt = cnt_ref[0, 0, 0] + cnt_ref[1, 0, 0]
    mean = sum_ref[...] / jnp.maximum(cnt, 1.0)[:, None]
    y = _mm_t(mean, wl_ref[...]) + _mm_t(x_ref[...], wr_ref[...]) \
        + b_ref[...][None, :]
    if relu:
        y = jnp.maximum(y, 0.0)
    o_ref[...] = y


def _update_dec_body(sum_ref, cnt_ref, x_ref, wl_ref, b_ref, wr_ref, wd_ref,
                     o_ref):
    cnt = cnt_ref[0, 0, 0] + cnt_ref[1, 0, 0]
    mean = sum_ref[...] / jnp.maximum(cnt, 1.0)[:, None]
    y = _mm_t(mean, wl_ref[...]) + _mm_t(x_ref[...], wr_ref[...]) \
        + b_ref[...][None, :]
    o_ref[...] = _mm_t(y, wd_ref[...])


def _full(shape):
    return pl.BlockSpec(shape, lambda i: tuple(0 for _ in shape))


def _update_tc(sum_x, cnt2, x, wl, b, wr, relu):
    cnt2 = cnt2.reshape(2, N // _BLK, 1, _BLK)
    return pl.pallas_call(
        functools.partial(_update_body, relu),
        grid=(N // _BLK,),
        in_specs=[
            pl.BlockSpec((_BLK, D), lambda i: (i, 0)),
            pl.BlockSpec((2, 1, 1, _BLK), lambda i: (0, i, 0, 0)),
            pl.BlockSpec((_BLK, D), lambda i: (i, 0)),
            _full((H, D)), _full((H,)), _full((H, D)),
        ],
        out_specs=pl.BlockSpec((_BLK, H), lambda i: (i, 0)),
        out_shape=jax.ShapeDtypeStruct((N, H), jnp.float32),
    )(sum_x, cnt2, x, wl, b, wr)


def _update_dec_tc(sum_x, cnt2, x, wl, b, wr, wd):
    cnt2 = cnt2.reshape(2, N // _BLK, 1, _BLK)
    return pl.pallas_call(
        _update_dec_body,
        grid=(N // _BLK,),
        in_specs=[
            pl.BlockSpec((_BLK, D), lambda i: (i, 0)),
            pl.BlockSpec((2, 1, 1, _BLK), lambda i: (0, i, 0, 0)),
            pl.BlockSpec((_BLK, D), lambda i: (i, 0)),
            _full((H, D)), _full((H,)), _full((H, D)), _full((H, H)),
        ],
        out_specs=pl.BlockSpec((_BLK, H), lambda i: (i, 0)),
        out_shape=jax.ShapeDtypeStruct((N, H), jnp.float32),
    )(sum_x, cnt2, x, wl, b, wr, wd)


def kernel(x_user, x_game, edge_index_plays, edge_index_rev, edge_score_index,
           params):
    del edge_index_rev
    pu = edge_index_plays[0].astype(jnp.int32)
    pg = edge_index_plays[1].astype(jnp.int32)
    sl8 = jnp.arange(8, dtype=jnp.int32)[:, None]
    src_rev = (pg[None, :] * 8 + sl8).reshape(-1)
    src_pl = (pu[None, :] * 8 + sl8).reshape(-1)
    dst_rev = pu
    dst_pl = pg

    counts = _make_counts()(pu, pg)
    cnt_u2 = counts[:, :5000]
    cnt_g2 = counts[:, 5000:]

    segsum = _make_segsum()
    u, g = x_user, x_game
    for l in (1, 2, 3):
        sums = segsum(g.reshape(N * NSL, LANES), u.reshape(N * NSL, LANES),
                      src_rev, dst_rev, src_pl, dst_pl)
        sum_u = sums[0].reshape(N, D)
        sum_g = sums[1].reshape(N, D)
        if l < 3:
            u_new = _update_tc(sum_u, cnt_u2, u, params['W%dr_l' % l],
                               params['b%dr' % l], params['W%dr_r' % l], True)
            g_new = _update_tc(sum_g, cnt_g2, g, params['W%dp_l' % l],
                               params['b%dp' % l], params['W%dp_r' % l], True)
            u, g = u_new, g_new
        else:
            p = _update_dec_tc(sum_u, cnt_u2, u, params['W3r_l'],
                               params['b3r'], params['W3r_r'],
                               params['Wd1'][:, :H])
            q = _update_dec_tc(sum_g, cnt_g2, g, params['W3p_l'],
                               params['b3p'], params['W3p_r'],
                               params['Wd1'][:, H:])

    row = edge_score_index[0].astype(jnp.int32)
    col = edge_score_index[1].astype(jnp.int32)
    bd2p = jnp.zeros((LANES,), jnp.float32).at[0].set(params['bd2'][0])
    out = _make_decoder()(p, q, row, col, params['bd1'],
                          params['Wd2'].reshape(-1), bd2p)
    return out


# speedup vs baseline: 4.3432x; 1.7503x over previous
import functools

import jax
import jax.numpy as jnp
from jax import lax
from jax.experimental import pallas as pl
from jax.experimental.pallas import tpu as pltpu
from jax.experimental.pallas import tpu_sc as plsc

N_USERS = 5000
N_GAMES = 5000
N = 5000
D = 128
H = 128
E = 320000
NSCORE = 100000

NSL = 8
LANES = 16
SEG_C = 800
SEG_HALF = E // 2
SEG_NCH = SEG_HALF // SEG_C
RED_C = 500

CNT_C = 2000
CNT_PER_TILE = E // 32
CNT_NCH = CNT_PER_TILE // CNT_C

DEC_C = 400
DEC_NCH = NSCORE // DEC_C


def _seg_mesh():
    return plsc.VectorSubcoreMesh(core_axis_name="c", subcore_axis_name="s")


def _make_counts():
    def body(u_hbm, g_hbm, out_hbm, ubuf, gbuf, ubuf_f, cnt, shared, sem):
        c = lax.axis_index("c")
        s = lax.axis_index("s")
        wid = c * 16 + s
        ones = jnp.ones((LANES,), jnp.float32)

        @plsc.parallel_loop(0, 10000 // LANES, 1, unroll=8)
        def zbody(i):
            cnt[pl.ds(i * LANES, LANES)] = jnp.zeros((LANES,), jnp.float32)

        base_tile = wid * CNT_PER_TILE

        def chunk(k, _):
            b = base_tile + k * CNT_C
            pltpu.sync_copy(u_hbm.at[pl.ds(b, CNT_C)], ubuf)
            pltpu.sync_copy(g_hbm.at[pl.ds(b, CNT_C)], gbuf)

            @plsc.parallel_loop(0, CNT_C // LANES, 1, unroll=4)
            def grp(j):
                uvec = ubuf[pl.ds(j * LANES, LANES)]
                gvec = gbuf[pl.ds(j * LANES, LANES)]
                plsc.addupdate_scatter(cnt, [uvec], ones)
                plsc.addupdate_scatter(cnt, [gvec + 5000], ones)
            return 0
        lax.fori_loop(0, CNT_NCH, chunk, 0)

        pltpu.sync_copy(cnt, shared.at[s])
        plsc.subcore_barrier()

        @pl.when(s == 0)
        def _():
            def partner(p, _):
                def pchunk(q, _):
                    pltpu.sync_copy(shared.at[p, pl.ds(q * 2000, 2000)], ubuf_f)

                    qb = q * 2000

                    @plsc.parallel_loop(0, 2000 // LANES, 1, unroll=8)
                    def radd(r):
                        plsc.addupdate(
                            cnt.at[pl.ds(qb + r * LANES, LANES)],
                            ubuf_f[pl.ds(r * LANES, LANES)])
                    return 0
                lax.fori_loop(0, 5, pchunk, 0)
                return 0
            lax.fori_loop(1, 16, partner, 0)
            pltpu.sync_copy(cnt, out_hbm.at[c])
        del sem

    return pl.kernel(
        body,
        out_type=jax.ShapeDtypeStruct((2, 10000), jnp.float32),
        mesh=_seg_mesh(),
        compiler_params=pltpu.CompilerParams(use_tc_tiling_on_sc=False, needs_layout_passes=False),
        scratch_types=[
            pltpu.VMEM((CNT_C,), jnp.int32),
            pltpu.VMEM((CNT_C,), jnp.int32),
            pltpu.VMEM((2000,), jnp.float32),
            pltpu.VMEM((10000,), jnp.float32),
            pltpu.VMEM_SHARED((16, 10000), jnp.float32),
            pltpu.SemaphoreType.DMA,
        ],
    )


def _segsum_body(xg_hbm, xu_hbm, src_rev, dst_rev, src_pl, dst_pl, out_hbm,
                 sbuf0, sbuf1, sbuf2, dbuf0, dbuf1, dbuf2,
                 gath0, gath1, gath2, acc, shared,
                 sem_i0, sem_i1, sem_i2, sem_g0, sem_g1, sem_g2):
    c = lax.axis_index("c")
    s = lax.axis_index("s")
    slc = s // 2
    half = s % 2
    lane = lax.iota(jnp.int32, LANES)
    colperm = [(lane + i) % LANES for i in range(LANES)]

    @plsc.parallel_loop(0, N, 1, unroll=8)
    def zbody(i):
        acc[i] = jnp.zeros((LANES,), jnp.float32)

    ebase0 = half * SEG_HALF

    def run_dir(table, src_sl, dst_e):
        sbufs = (sbuf0, sbuf1, sbuf2)
        dbufs = (dbuf0, dbuf1, dbuf2)
        gaths = (gath0, gath1, gath2)
        sems_i = (sem_i0, sem_i1, sem_i2)
        sems_g = (sem_g0, sem_g1, sem_g2)

        sbase = slc * E + half * SEG_HALF

        def idx_start(k, o):
            pltpu.async_copy(src_sl.at[pl.ds(sbase + k * SEG_C, SEG_C)],
                             sbufs[o], sems_i[o])
            pltpu.async_copy(dst_e.at[pl.ds(ebase0 + k * SEG_C, SEG_C)],
                             dbufs[o], sems_i[o])

        def idx_wait(o):
            pltpu.make_async_copy(src_sl.at[pl.ds(0, SEG_C)], sbufs[o],
                                  sems_i[o]).wait()
            pltpu.make_async_copy(dst_e.at[pl.ds(0, SEG_C)], dbufs[o],
                                  sems_i[o]).wait()

        def gather_start(o):
            pltpu.async_copy(table.at[sbufs[o]], gaths[o], sems_g[o])

        def gather_wait(o):
            pltpu.make_async_copy(table.at[sbufs[o]], gaths[o],
                                  sems_g[o]).wait()

        def accum(gath, dbuf):
            @plsc.parallel_loop(0, SEG_C // LANES, 1, unroll=2)
            def grp(j):
                dvec = dbuf[pl.ds(j * LANES, LANES)]
                evec = lane + j * LANES
                for i in range(LANES):
                    colv = colperm[i]
                    vals = plsc.load_gather(gath, [evec, colv])
                    plsc.addupdate_scatter(acc, [dvec, colv], vals)

        pltpu.sync_copy(src_sl.at[pl.ds(sbase, SEG_C)], sbuf0)
        pltpu.sync_copy(dst_e.at[pl.ds(ebase0, SEG_C)], dbuf0)
        pltpu.sync_copy(src_sl.at[pl.ds(sbase + SEG_C, SEG_C)], sbuf1)
        pltpu.sync_copy(dst_e.at[pl.ds(ebase0 + SEG_C, SEG_C)], dbuf1)
        gather_start(0)
        gather_start(1)
        idx_start(2, 2)

        def triple(kk, _):
            for o in range(3):
                k = kk * 3 + o

                @pl.when(k + 2 < SEG_NCH)
                def _():
                    idx_wait((o + 2) % 3)
                    gather_start((o + 2) % 3)

                @pl.when(k < SEG_NCH)
                def _():
                    gather_wait(o)
                    accum(gaths[o], dbufs[o])

                @pl.when(k + 3 < SEG_NCH)
                def _():
                    idx_start(k + 3, o)
            return 0
        lax.fori_loop(0, (SEG_NCH + 2) // 3, triple, 0)

    @pl.when(c == 0)
    def _():
        run_dir(xg_hbm, src_rev, dst_rev)

    @pl.when(c == 1)
    def _():
        run_dir(xu_hbm, src_pl, dst_pl)

    def pchunk(q, _):
        @pl.when(half == 1)
        def _():
            pltpu.sync_copy(acc.at[pl.ds(q * RED_C, RED_C)], shared.at[slc])
        plsc.subcore_barrier()

        @pl.when(half == 0)
        def _():
            pltpu.sync_copy(shared.at[slc], gath0.at[pl.ds(0, RED_C)])

            qb = q * RED_C

            @plsc.parallel_loop(0, RED_C, 1, unroll=8)
            def radd(r):
                plsc.addupdate(acc.at[qb + r], gath0[r])
        plsc.subcore_barrier()
        return 0
    lax.fori_loop(0, N // RED_C, pchunk, 0)

    @pl.when(half == 0)
    def _():
        pltpu.sync_copy(acc, out_hbm.at[c, :, slc, :])


def _make_segsum():
    return pl.kernel(
        _segsum_body,
        out_type=jax.ShapeDtypeStruct((2, N, NSL, LANES), jnp.float32),
        mesh=_seg_mesh(),
        compiler_params=pltpu.CompilerParams(use_tc_tiling_on_sc=False, needs_layout_passes=False),
        scratch_types=[
            pltpu.VMEM((SEG_C,), jnp.int32),
            pltpu.VMEM((SEG_C,), jnp.int32),
            pltpu.VMEM((SEG_C,), jnp.int32),
            pltpu.VMEM((SEG_C,), jnp.int32),
            pltpu.VMEM((SEG_C,), jnp.int32),
            pltpu.VMEM((SEG_C,), jnp.int32),
            pltpu.VMEM((SEG_C, LANES), jnp.float32),
            pltpu.VMEM((SEG_C, LANES), jnp.float32),
            pltpu.VMEM((SEG_C, LANES), jnp.float32),
            pltpu.VMEM((N, LANES), jnp.float32),
            pltpu.VMEM_SHARED((NSL, RED_C, LANES), jnp.float32),
            pltpu.SemaphoreType.DMA,
            pltpu.SemaphoreType.DMA,
            pltpu.SemaphoreType.DMA,
            pltpu.SemaphoreType.DMA,
            pltpu.SemaphoreType.DMA,
            pltpu.SemaphoreType.DMA,
        ],
    )


def _decoder_body(p_hbm, q_hbm, row_hbm, col_hbm, bd1_hbm, wd2_hbm, bd2_hbm,
                  out_hbm, rbuf, cbuf, buf, obuf, bd1v, wd2v, bd2v, sem):
    c = lax.axis_index("c")
    s = lax.axis_index("s")
    wid = s * 2 + c
    lane = lax.iota(jnp.int32, LANES)
    last_mask = lane == (LANES - 1)

    pltpu.sync_copy(bd1_hbm, bd1v)
    pltpu.sync_copy(wd2_hbm, wd2v)
    pltpu.sync_copy(bd2_hbm, bd2v)

    bd2vec = bd2v[pl.ds(0, LANES)]
    b1 = [bd1v[pl.ds(f * LANES, LANES)] for f in range(NSL)]
    w2 = [wd2v[pl.ds(f * LANES, LANES)] for f in range(NSL)]

    def chunk(k, _):
        jc = wid + 32 * k

        @pl.when(jc < DEC_NCH)
        def _():
            b = jc * DEC_C
            pltpu.sync_copy(row_hbm.at[pl.ds(b, DEC_C)], rbuf)
            pltpu.sync_copy(col_hbm.at[pl.ds(b, DEC_C)], cbuf)
            pltpu.async_copy(p_hbm.at[rbuf], buf, sem).wait()
            pltpu.async_copy(q_hbm.at[cbuf], buf, sem, add=True).wait()

            @plsc.parallel_loop(0, DEC_C, 1, unroll=2)
            def edge(e):
                acc = bd2vec
                for f in range(NSL):
                    v = buf[e, pl.ds(f * LANES, LANES)]
                    acc = acc + jnp.maximum(v + b1[f], 0.0) * w2[f]
                tot = plsc.cumsum(acc)
                plsc.store_scatter(obuf, [jnp.broadcast_to(e, (LANES,))],
                                   tot, mask=last_mask)
            pltpu.sync_copy(obuf, out_hbm.at[pl.ds(b, DEC_C)])
        return 0

    lax.fori_loop(0, (DEC_NCH + 31) // 32, chunk, 0)


def _make_decoder():
    return pl.kernel(
        _decoder_body,
        out_type=jax.ShapeDtypeStruct((NSCORE,), jnp.float32),
        mesh=_seg_mesh(),
        compiler_params=pltpu.CompilerParams(use_tc_tiling_on_sc=False, needs_layout_passes=False),
        scratch_types=[
            pltpu.VMEM((DEC_C,), jnp.int32),
            pltpu.VMEM((DEC_C,), jnp.int32),
            pltpu.VMEM((DEC_C, D), jnp.float32),
            pltpu.VMEM((DEC_C,), jnp.float32),
            pltpu.VMEM((D,), jnp.float32),
            pltpu.VMEM((D,), jnp.float32),
            pltpu.VMEM((LANES,), jnp.float32),
            pltpu.SemaphoreType.DMA,
        ],
    )


_BLK = 1000


def _mm_t(a, w):
    return lax.dot_general(a, w, (((1,), (1,)), ((), ())),
                           preferred_element_type=jnp.float32)


def _update_body(relu, sum_ref, cnt_ref, x_ref, wl_ref, b_ref, wr_ref, o_ref):
    cnt = cnt_ref[0, 0, 0] + cnt_ref[1, 0, 0]
    mean = sum_ref[...] / jnp.maximum(cnt, 1.0)[:, None]
    y = _mm_t(mean, wl_ref[...]) + _mm_t(x_ref[...], wr_ref[...]) \
        + b_ref[...][None, :]
    if relu:
        y = jnp.maximum(y, 0.0)
    o_ref[...] = y


def _update_dec_body(sum_ref, cnt_ref, x_ref, wl_ref, b_ref, wr_ref, wd_ref,
                     o_ref):
    cnt = cnt_ref[0, 0, 0] + cnt_ref[1, 0, 0]
    mean = sum_ref[...] / jnp.maximum(cnt, 1.0)[:, None]
    y = _mm_t(mean, wl_ref[...]) + _mm_t(x_ref[...], wr_ref[...]) \
        + b_ref[...][None, :]
    o_ref[...] = _mm_t(y, wd_ref[...])


def _full(shape):
    return pl.BlockSpec(shape, lambda i: tuple(0 for _ in shape))


def _update_tc(sum_x, cnt2, x, wl, b, wr, relu):
    cnt2 = cnt2.reshape(2, N // _BLK, 1, _BLK)
    return pl.pallas_call(
        functools.partial(_update_body, relu),
        grid=(N // _BLK,),
        in_specs=[
            pl.BlockSpec((_BLK, D), lambda i: (i, 0)),
            pl.BlockSpec((2, 1, 1, _BLK), lambda i: (0, i, 0, 0)),
            pl.BlockSpec((_BLK, D), lambda i: (i, 0)),
            _full((H, D)), _full((H,)), _full((H, D)),
        ],
        out_specs=pl.BlockSpec((_BLK, H), lambda i: (i, 0)),
        out_shape=jax.ShapeDtypeStruct((N, H), jnp.float32),
    )(sum_x, cnt2, x, wl, b, wr)


def _update_dec_tc(sum_x, cnt2, x, wl, b, wr, wd):
    cnt2 = cnt2.reshape(2, N // _BLK, 1, _BLK)
    return pl.pallas_call(
        _update_dec_body,
        grid=(N // _BLK,),
        in_specs=[
            pl.BlockSpec((_BLK, D), lambda i: (i, 0)),
            pl.BlockSpec((2, 1, 1, _BLK), lambda i: (0, i, 0, 0)),
            pl.BlockSpec((_BLK, D), lambda i: (i, 0)),
            _full((H, D)), _full((H,)), _full((H, D)), _full((H, H)),
        ],
        out_specs=pl.BlockSpec((_BLK, H), lambda i: (i, 0)),
        out_shape=jax.ShapeDtypeStruct((N, H), jnp.float32),
    )(sum_x, cnt2, x, wl, b, wr, wd)


def kernel(x_user, x_game, edge_index_plays, edge_index_rev, edge_score_index,
           params):
    del edge_index_rev
    pu = edge_index_plays[0].astype(jnp.int32)
    pg = edge_index_plays[1].astype(jnp.int32)
    sl8 = jnp.arange(8, dtype=jnp.int32)[:, None]
    src_rev = (pg[None, :] * 8 + sl8).reshape(-1)
    src_pl = (pu[None, :] * 8 + sl8).reshape(-1)
    dst_rev = pu
    dst_pl = pg

    counts = _make_counts()(pu, pg)
    cnt_u2 = counts[:, :5000]
    cnt_g2 = counts[:, 5000:]

    segsum = _make_segsum()
    u, g = x_user, x_game
    for l in (1, 2, 3):
        sums = segsum(g.reshape(N * NSL, LANES), u.reshape(N * NSL, LANES),
                      src_rev, dst_rev, src_pl, dst_pl)
        sum_u = sums[0].reshape(N, D)
        sum_g = sums[1].reshape(N, D)
        if l < 3:
            u_new = _update_tc(sum_u, cnt_u2, u, params['W%dr_l' % l],
                               params['b%dr' % l], params['W%dr_r' % l], True)
            g_new = _update_tc(sum_g, cnt_g2, g, params['W%dp_l' % l],
                               params['b%dp' % l], params['W%dp_r' % l], True)
            u, g = u_new, g_new
        else:
            p = _update_dec_tc(sum_u, cnt_u2, u, params['W3r_l'],
                               params['b3r'], params['W3r_r'],
                               params['Wd1'][:, :H])
            q = _update_dec_tc(sum_g, cnt_g2, g, params['W3p_l'],
                               params['b3p'], params['W3p_r'],
                               params['Wd1'][:, H:])

    row = edge_score_index[0].astype(jnp.int32)
    col = edge_score_index[1].astype(jnp.int32)
    bd2p = jnp.zeros((LANES,), jnp.float32).at[0].set(params['bd2'][0])
    out = _make_decoder()(p, q, row, col, params['bd1'],
                          params['Wd2'].reshape(-1), bd2p)
    return out
